# baseline (speedup 1.0000x reference)
import functools

import jax
import jax.numpy as jnp
from jax import lax
from jax.experimental import pallas as pl
from jax.experimental.pallas import tpu as pltpu
from jax.experimental.pallas import tpu_sc as plsc

_CHUNK = 128
_SUPER = 2
_NBUF = 2


@functools.cache
def _build(n, vocab, d):
    info = plsc.get_sparse_core_info()
    nc, ns = info.num_cores, info.num_subcores
    nw = nc * ns
    per_w = n // nw
    n_chunks = per_w // _CHUNK
    n_super = n_chunks // _SUPER
    assert per_w * nw == n and n_chunks * _CHUNK == per_w
    assert n_super * _SUPER == n_chunks
    assert n_super % _NBUF == 0 and n_super // _NBUF >= 2

    mesh = plsc.VectorSubcoreMesh(core_axis_name="c", subcore_axis_name="s")

    @functools.partial(
        pl.kernel,
        mesh=mesh,
        out_type=jax.ShapeDtypeStruct((n, d), jnp.float32),
        scratch_types=[
            pltpu.VMEM((n_chunks, _CHUNK), jnp.int32),
            [pltpu.VMEM((_SUPER * _CHUNK, d), jnp.float32)] * _NBUF,
            [pltpu.SemaphoreType.DMA] * _NBUF,
            [pltpu.SemaphoreType.DMA] * _NBUF,
        ],
    )
    def k(idx_hbm, table_hbm, out_hbm, idx_v, rows, gsem, osem):
        wid = lax.axis_index("s") * nc + lax.axis_index("c")
        crow = wid * n_chunks
        base = wid * per_w
        pltpu.sync_copy(idx_hbm.at[pl.ds(crow, n_chunks)], idx_v)

        def gathers(sj, b):
            return [
                pltpu.make_async_copy(
                    table_hbm.at[idx_v.at[sj * _SUPER + h]],
                    rows[b].at[pl.ds(h * _CHUNK, _CHUNK)],
                    gsem[b],
                )
                for h in range(_SUPER)
            ]

        def writeback(sj, b):
            return pltpu.make_async_copy(
                rows[b],
                out_hbm.at[pl.ds(base + sj * _SUPER * _CHUNK, _SUPER * _CHUNK)],
                osem[b],
            )

        for b in range(_NBUF):
            for g in gathers(b, b):
                g.start()

        def body(k_it, carry):
            sj = k_it * _NBUF
            for b in range(_NBUF):
                for g in gathers(sj + b, b):
                    g.wait()
                writeback(sj + b, b).start()
                writeback(sj + b, b).wait()
                for g in gathers(sj + b + _NBUF, b):
                    g.start()
            return carry

        lax.fori_loop(0, n_super // _NBUF - 1, body, 0)

        sj0 = n_super - _NBUF
        for b in range(_NBUF):
            for g in gathers(sj0 + b, b):
                g.wait()
            writeback(sj0 + b, b).start()
        for b in range(_NBUF):
            writeback(sj0 + b, b).wait()

    return k


def kernel(inp, table):
    b, s = inp.shape
    vocab, d = table.shape
    n = b * s
    idx2d = inp.reshape(n // _CHUNK, _CHUNK)
    out = _build(n, vocab, d)(idx2d, table)
    return out.reshape(b, s, d)


# speedup vs baseline: 1.0026x; 1.0026x over previous
import functools

import jax
import jax.numpy as jnp
from jax import lax
from jax.experimental import pallas as pl
from jax.experimental.pallas import tpu as pltpu
from jax.experimental.pallas import tpu_sc as plsc

_CHUNK = 128
_NBUF = 5


@functools.cache
def _build(n, vocab, d):
    info = plsc.get_sparse_core_info()
    nc, ns = info.num_cores, info.num_subcores
    nw = nc * ns
    per_w = n // nw
    n_chunks = per_w // _CHUNK
    assert per_w * nw == n and n_chunks * _CHUNK == per_w
    assert n_chunks % _NBUF == 0 and n_chunks // _NBUF >= 2

    mesh = plsc.VectorSubcoreMesh(core_axis_name="c", subcore_axis_name="s")

    @functools.partial(
        pl.kernel,
        mesh=mesh,
        out_type=jax.ShapeDtypeStruct((n, d), jnp.float32),
        scratch_types=[
            pltpu.VMEM((n_chunks, _CHUNK), jnp.int32),
            [pltpu.VMEM((_CHUNK, d), jnp.float32)] * _NBUF,
            [pltpu.SemaphoreType.DMA] * _NBUF,
            [pltpu.SemaphoreType.DMA] * _NBUF,
        ],
    )
    def k(idx_hbm, table_hbm, out_hbm, idx_v, rows, gsem, osem):
        wid = lax.axis_index("s") * nc + lax.axis_index("c")
        crow = wid * n_chunks
        base = wid * per_w
        pltpu.sync_copy(idx_hbm.at[pl.ds(crow, n_chunks)], idx_v)

        def gather(j, b):
            return pltpu.make_async_copy(
                table_hbm.at[idx_v.at[j]], rows[b], gsem[b]
            )

        def writeback(j, b):
            return pltpu.make_async_copy(
                rows[b], out_hbm.at[pl.ds(base + j * _CHUNK, _CHUNK)], osem[b]
            )

        for b in range(_NBUF):
            gather(b, b).start()

        def body(k_it, carry):
            j0 = k_it * _NBUF
            for b in range(_NBUF):
                j = j0 + b
                gather(j, b).wait()
                writeback(j, b).start()
                writeback(j, b).wait()
                gather(j + _NBUF, b).start()
            return carry

        lax.fori_loop(0, n_chunks // _NBUF - 1, body, 0)

        j0 = n_chunks - _NBUF
        for b in range(_NBUF):
            j = j0 + b
            gather(j, b).wait()
            writeback(j, b).start()
        for b in range(_NBUF):
            writeback(j0 + b, b).wait()

    return k


def kernel(inp, table):
    b, s = inp.shape
    vocab, d = table.shape
    n = b * s
    idx2d = inp.reshape(n // _CHUNK, _CHUNK)
    out = _build(n, vocab, d)(idx2d, table)
    return out.reshape(b, s, d)
